# baseline (speedup 1.0000x reference)
import functools
import numpy as np
import jax
import jax.numpy as jnp
from jax.experimental import pallas as pl
from jax.experimental.pallas import tpu as pltpu

B = 2
S = 2048
D = 1024
H = 16
DK = 64
W = 8
MW = 2 * W + 1
EPS = 1e-6

BQ = 256
NB = S // BQ
HALO = BQ + 2 * W
_NEG = -1e30


def _band_tables():
    i = np.arange(BQ)[:, None]
    j = np.arange(HALO)[None, :]
    bias = np.empty((3, BQ, HALO), np.float32)
    ninv = np.empty((3, BQ, 1), np.float32)
    for t, (r0, start) in enumerate([(0, 0), (BQ, BQ - W), (S - BQ, S - HALO)]):
        band = np.abs((start + j) - (r0 + i)) <= W
        bias[t] = np.where(band, 0.0, _NEG)
        ninv[t] = MW - band.sum(axis=1, keepdims=True)
    return jnp.asarray(bias), jnp.asarray(ninv)


def _fused_kernel(hs_ref, wq_ref, wk_ref, wv_ref, wf_ref, bf_ref, g_ref,
                  bt_ref, bias_ref, ninv_ref, out_ref):
    b = pl.program_id(0)
    blk = pl.program_id(1)
    r0 = blk * BQ

    x = hs_ref[b, pl.ds(r0, BQ), :]
    start = jnp.clip(r0 - W, 0, S - HALO)
    xh = hs_ref[b, pl.ds(pl.multiple_of(start, 8), HALO), :]
    xb = x.astype(jnp.bfloat16)
    xhb = xh.astype(jnp.bfloat16)

    qb = jnp.dot(xb, wq_ref[...],
                 preferred_element_type=jnp.float32).astype(jnp.bfloat16)
    kb = jnp.dot(xhb, wk_ref[...],
                 preferred_element_type=jnp.float32).astype(jnp.bfloat16)
    vb = jnp.dot(xhb, wv_ref[...],
                 preferred_element_type=jnp.float32).astype(jnp.bfloat16)

    tsel = jnp.where(blk == 0, 0, jnp.where(blk == NB - 1, 2, 1))
    bias = bias_ref[tsel]
    n_inv = ninv_ref[tsel]

    outs = []
    for h in range(H):
        qh = qb[:, h * DK:(h + 1) * DK]
        kh = kb[:, h * DK:(h + 1) * DK]
        vh = vb[:, h * DK:(h + 1) * DK]
        sh = jax.lax.dot_general(qh, kh, (((1,), (1,)), ((), ())),
                                 preferred_element_type=jnp.float32)
        e = jnp.exp(sh + bias)
        denom = jnp.sum(e, axis=1, keepdims=True) + n_inv
        o = jax.lax.dot_general(e.astype(jnp.bfloat16), vh,
                                (((1,), (0,)), ((), ())),
                                preferred_element_type=jnp.float32)
        outs.append(o / denom)
    attn = jnp.concatenate(outs, axis=-1)

    ctx = jnp.dot(attn.astype(jnp.bfloat16), wf_ref[...],
                  preferred_element_type=jnp.float32)
    ctx = ctx + bf_ref[...] + x
    mean = jnp.mean(ctx, axis=1, keepdims=True)
    cen = ctx - mean
    var = jnp.mean(cen * cen, axis=1, keepdims=True)
    out_ref[0] = cen * jax.lax.rsqrt(var + EPS) * g_ref[...] + bt_ref[...]


def kernel(hidden_states, w_qs, w_ks, w_vs, w_fc, b_fc, gamma, beta, q_k_mask):
    del q_k_mask
    bias_tab, ninv_tab = _band_tables()
    full = lambda shape: pl.BlockSpec(shape, lambda b, i: (0,) * len(shape))
    return pl.pallas_call(
        _fused_kernel,
        grid=(B, NB),
        in_specs=[
            full((B, S, D)),
            full((D, D)),
            full((D, D)),
            full((D, D)),
            full((D, D)),
            full((1, D)),
            full((1, D)),
            full((1, D)),
            full((3, BQ, HALO)),
            full((3, BQ, 1)),
        ],
        out_specs=pl.BlockSpec((1, BQ, D), lambda b, i: (b, i, 0)),
        out_shape=jax.ShapeDtypeStruct((B, S, D), jnp.float32),
    )(hidden_states,
      (w_qs * np.float32(1.0 / np.sqrt(DK))).astype(jnp.bfloat16),
      w_ks.astype(jnp.bfloat16),
      w_vs.astype(jnp.bfloat16), w_fc.astype(jnp.bfloat16),
      b_fc.reshape(1, D), gamma.reshape(1, D), beta.reshape(1, D),
      bias_tab, ninv_tab)


# speedup vs baseline: 1.0051x; 1.0051x over previous
import functools
import numpy as np
import jax
import jax.numpy as jnp
from jax.experimental import pallas as pl
from jax.experimental.pallas import tpu as pltpu

B = 2
S = 2048
D = 1024
H = 16
DK = 64
W = 8
MW = 2 * W + 1
EPS = 1e-6

BQ = 256
NB = S // BQ
HALO = BQ + 2 * W
_NEG = -1e30


def _band_tables():
    i = np.arange(BQ)[:, None]
    j = np.arange(HALO)[None, :]
    bias = np.empty((NB, BQ, HALO), np.float32)
    ninv = np.empty((NB, BQ, 1), np.float32)
    for t in range(NB):
        r0 = t * BQ
        start = min(max(r0 - W, 0), S - HALO)
        band = np.abs((start + j) - (r0 + i)) <= W
        bias[t] = np.where(band, 0.0, _NEG)
        ninv[t] = MW - band.sum(axis=1, keepdims=True)
    return jnp.asarray(bias), jnp.asarray(ninv)


def _fused_kernel(hs_ref, wq_ref, wk_ref, wv_ref, wf_ref, bf_ref, g_ref,
                  bt_ref, bias_ref, ninv_ref, out_ref):
    b = pl.program_id(0)
    blk = pl.program_id(1)
    r0 = blk * BQ

    x = hs_ref[b, pl.ds(r0, BQ), :]
    start = jnp.clip(r0 - W, 0, S - HALO)
    xh = hs_ref[b, pl.ds(pl.multiple_of(start, 8), HALO), :]
    xb = x.astype(jnp.bfloat16)
    xhb = xh.astype(jnp.bfloat16)

    qb = jnp.dot(xb, wq_ref[...],
                 preferred_element_type=jnp.float32).astype(jnp.bfloat16)
    kb = jnp.dot(xhb, wk_ref[...],
                 preferred_element_type=jnp.float32).astype(jnp.bfloat16)
    vb = jnp.dot(xhb, wv_ref[...],
                 preferred_element_type=jnp.float32).astype(jnp.bfloat16)

    bias = bias_ref[0]
    n_inv = ninv_ref[0]

    outs = []
    for h in range(H):
        qh = qb[:, h * DK:(h + 1) * DK]
        kh = kb[:, h * DK:(h + 1) * DK]
        vh = vb[:, h * DK:(h + 1) * DK]
        sh = jax.lax.dot_general(qh, kh, (((1,), (1,)), ((), ())),
                                 preferred_element_type=jnp.float32)
        e = jnp.exp(sh + bias)
        denom = jnp.sum(e, axis=1, keepdims=True) + n_inv
        o = jax.lax.dot_general(e.astype(jnp.bfloat16), vh,
                                (((1,), (0,)), ((), ())),
                                preferred_element_type=jnp.float32)
        outs.append(o / denom)
    attn = jnp.concatenate(outs, axis=-1)

    ctx = jnp.dot(attn.astype(jnp.bfloat16), wf_ref[...],
                  preferred_element_type=jnp.float32)
    ctx = ctx + bf_ref[...] + x
    mean = jnp.mean(ctx, axis=1, keepdims=True)
    cen = ctx - mean
    var = jnp.mean(cen * cen, axis=1, keepdims=True)
    out_ref[0] = cen * jax.lax.rsqrt(var + EPS) * g_ref[...] + bt_ref[...]


def kernel(hidden_states, w_qs, w_ks, w_vs, w_fc, b_fc, gamma, beta, q_k_mask):
    del q_k_mask
    bias_tab, ninv_tab = _band_tables()
    full = lambda shape: pl.BlockSpec(shape, lambda b, i: (0,) * len(shape))
    return pl.pallas_call(
        _fused_kernel,
        grid=(B, NB),
        in_specs=[
            full((B, S, D)),
            full((D, D)),
            full((D, D)),
            full((D, D)),
            full((D, D)),
            full((1, D)),
            full((1, D)),
            full((1, D)),
            pl.BlockSpec((1, BQ, HALO), lambda b, i: (i, 0, 0)),
            pl.BlockSpec((1, BQ, 1), lambda b, i: (i, 0, 0)),
        ],
        out_specs=pl.BlockSpec((1, BQ, D), lambda b, i: (b, i, 0)),
        out_shape=jax.ShapeDtypeStruct((B, S, D), jnp.float32),
    )(hidden_states,
      (w_qs * np.float32(1.0 / np.sqrt(DK))).astype(jnp.bfloat16),
      w_ks.astype(jnp.bfloat16),
      w_vs.astype(jnp.bfloat16), w_fc.astype(jnp.bfloat16),
      b_fc.reshape(1, D), gamma.reshape(1, D), beta.reshape(1, D),
      bias_tab, ninv_tab)


# speedup vs baseline: 1.1876x; 1.1816x over previous
import numpy as np
import jax
import jax.numpy as jnp
from jax.experimental import pallas as pl
from jax.experimental.pallas import tpu as pltpu

B = 2
S = 2048
D = 1024
H = 16
DK = 64
W = 8
EPS = 1e-6

BQ = 240
HALO = BQ + 2 * W
NB = -(-S // BQ)
SP = W + NB * BQ + W
_NEG = -1e30


def _band_bias():
    i = np.arange(BQ)[:, None]
    j = np.arange(HALO)[None, :]
    d = j - i
    return jnp.asarray(np.where((d >= 0) & (d <= 2 * W), 0.0, _NEG),
                       dtype=np.float32)


def _fused_kernel(hs_ref, wq_ref, wk_ref, wv_ref, wf_ref, bf_ref, g_ref,
                  bt_ref, bias_ref, out_ref):
    t = pl.program_id(0)
    q0 = W + t * BQ
    k0 = t * BQ

    x = jnp.concatenate(
        [hs_ref[b, pl.ds(pl.multiple_of(q0, 8), BQ), :] for b in range(B)],
        axis=0)
    xh = jnp.concatenate(
        [hs_ref[b, pl.ds(pl.multiple_of(k0, 8), HALO), :] for b in range(B)],
        axis=0)
    xb = x.astype(jnp.bfloat16)
    xhb = xh.astype(jnp.bfloat16)

    qb = jnp.dot(xb, wq_ref[...],
                 preferred_element_type=jnp.float32).astype(jnp.bfloat16)
    kb = jnp.dot(xhb, wk_ref[...],
                 preferred_element_type=jnp.float32).astype(jnp.bfloat16)
    vb = jnp.dot(xhb, wv_ref[...],
                 preferred_element_type=jnp.float32).astype(jnp.bfloat16)

    bias = bias_ref[...]
    ones_col = jnp.ones((HALO, 1), jnp.bfloat16)

    outs = []
    for b in range(B):
        qB = qb[b * BQ:(b + 1) * BQ]
        kB = kb[b * HALO:(b + 1) * HALO]
        vB = vb[b * HALO:(b + 1) * HALO]
        for h in range(H):
            qh = qB[:, h * DK:(h + 1) * DK]
            kh = kB[:, h * DK:(h + 1) * DK]
            vh = jnp.concatenate(
                [vB[:, h * DK:(h + 1) * DK], ones_col], axis=1)
            sh = jax.lax.dot_general(qh, kh, (((1,), (1,)), ((), ())),
                                     preferred_element_type=jnp.float32)
            e = jnp.exp(sh + bias)
            o = jax.lax.dot_general(e.astype(jnp.bfloat16), vh,
                                    (((1,), (0,)), ((), ())),
                                    preferred_element_type=jnp.float32)
            outs.append(o[:, :DK] / o[:, DK:DK + 1])
    attn = jnp.concatenate(
        [jnp.concatenate(outs[b * H:(b + 1) * H], axis=1) for b in range(B)],
        axis=0)

    ctx = jnp.dot(attn.astype(jnp.bfloat16), wf_ref[...],
                  preferred_element_type=jnp.float32)
    ctx = ctx + bf_ref[...] + x
    mean = jnp.mean(ctx, axis=1, keepdims=True)
    cen = ctx - mean
    var = jnp.mean(cen * cen, axis=1, keepdims=True)
    y = cen * jax.lax.rsqrt(var + EPS) * g_ref[...] + bt_ref[...]
    out_ref[0] = y[:BQ]
    out_ref[1] = y[BQ:]


def kernel(hidden_states, w_qs, w_ks, w_vs, w_fc, b_fc, gamma, beta, q_k_mask):
    del q_k_mask
    hs_pad = jnp.pad(hidden_states, ((0, 0), (W, SP - S - W), (0, 0)))
    full = lambda shape: pl.BlockSpec(shape, lambda t: (0,) * len(shape))
    out = pl.pallas_call(
        _fused_kernel,
        grid=(NB,),
        in_specs=[
            full((B, SP, D)),
            full((D, D)),
            full((D, D)),
            full((D, D)),
            full((D, D)),
            full((1, D)),
            full((1, D)),
            full((1, D)),
            full((BQ, HALO)),
        ],
        out_specs=pl.BlockSpec((B, BQ, D), lambda t: (0, t, 0)),
        out_shape=jax.ShapeDtypeStruct((B, NB * BQ, D), jnp.float32),
    )(hs_pad,
      (w_qs * np.float32(1.0 / np.sqrt(DK))).astype(jnp.bfloat16),
      w_ks.astype(jnp.bfloat16),
      w_vs.astype(jnp.bfloat16), w_fc.astype(jnp.bfloat16),
      b_fc.reshape(1, D), gamma.reshape(1, D), beta.reshape(1, D),
      _band_bias())
    return out[:, :S, :]
